# baseline (speedup 1.0000x reference)
import functools

import jax
import jax.numpy as jnp
from jax import lax
from jax.experimental import pallas as pl
from jax.experimental.pallas import tpu as pltpu
from jax.experimental.pallas import tpu_sc as plsc

K = 32
BLOCK_ROWS = 256

NUM_WORKERS = 32
BATCH = 16
NGROUPS = 32
CAP = 256
MAX_BISECT = 33
LATENT_DIM_STATIC = 3072


def _enc_block(x_ref, b_pre_ref, w_enc_ref, b_enc_ref, lat_ref):
    x_c = x_ref[...] - b_pre_ref[...]
    lat_ref[...] = jax.lax.dot_general(
        x_c, w_enc_ref[...], (((1,), (1,)), ((), ())),
        preferred_element_type=jnp.float32,
    ) + b_enc_ref[...]


def _dec_block(lat_ref, t_ref, b_pre_ref, w_dec_ref, b_dec_ref,
               recon_ref, sparse_ref):
    lat = lat_ref[...]
    t = t_ref[...]
    sparse = jnp.where(lat >= t, lat, 0.0)
    sparse_ref[...] = sparse
    recon_ref[...] = jax.lax.dot_general(
        sparse, w_dec_ref[...], (((1,), (1,)), ((), ())),
        preferred_element_type=jnp.float32,
    ) + b_dec_ref[...] + b_pre_ref[...]


def _to_sortable(v):
    iv = lax.bitcast_convert_type(v, jnp.int32)
    m = iv >> 31
    s = iv ^ (m | jnp.int32(-2147483648))
    return lax.bitcast_convert_type(s, jnp.uint32)


def _from_sortable(s):
    si = lax.bitcast_convert_type(s, jnp.int32)
    neg = si >= 0
    b = jnp.where(neg, ~si, si ^ jnp.int32(-2147483648))
    return lax.bitcast_convert_type(b, jnp.float32)


def _sc_thresh_body(lat_hbm, out_hbm, buf, cand, tstage):
    latent_dim = LATENT_DIM_STATIC
    n = lat_hbm.shape[0] // latent_dim
    rows_per_worker = n // NUM_WORKERS
    nbatch = rows_per_worker // BATCH
    gsize = latent_dim // NGROUPS

    wid = lax.axis_index("s") * 2 + lax.axis_index("c")
    base = wid * rows_per_worker

    lanes = lax.iota(jnp.int32, BATCH)
    row_off = lanes * latent_dim
    neg_inf = jnp.full((BATCH,), -jnp.inf, dtype=jnp.float32)

    def gather_col(col):
        return plsc.load_gather(buf, [row_off + col])

    def bisect(lo_u, hi_u, count_fn):
        t0 = jnp.zeros((BATCH,), jnp.float32)
        done0 = jnp.zeros((BATCH,), jnp.bool_)

        def cond(state):
            i, lo, hi, t, done = state
            ndone = jnp.max(plsc.all_reduce_population_count(done))
            return jnp.logical_and(i < MAX_BISECT, ndone < BATCH)

        def body(state):
            i, lo, hi, t, done = state
            mid_u = lo + ((hi - lo) >> jnp.uint32(1))
            mid_f = _from_sortable(mid_u)
            cnt = count_fn(mid_f)
            ge = cnt >= K
            eq = cnt == K
            upd = jnp.logical_not(done)
            t = jnp.where(jnp.logical_and(upd, eq), mid_f, t)
            done = jnp.logical_or(done, eq)
            lo = jnp.where(jnp.logical_and(upd, ge), mid_u, lo)
            hi = jnp.where(jnp.logical_and(upd, jnp.logical_not(ge)), mid_u, hi)
            return (i + 1, lo, hi, t, done)

        _, lo, hi, t, done = lax.while_loop(
            cond, body, (jnp.int32(0), lo_u, hi_u, t0, done0))
        return jnp.where(done, t, _from_sortable(lo))

    def process_batch(b, _):
        r0 = base + b * BATCH
        pltpu.sync_copy(lat_hbm.at[pl.ds(r0 * latent_dim, BATCH * latent_dim)],
                        buf)

        def group_body(g, la_ua):
            La, Ua = la_ua

            def col_body(j, gm):
                return jnp.maximum(gm, gather_col(g * gsize + j))

            gmax = lax.fori_loop(0, gsize, col_body, neg_inf)
            return (jnp.minimum(La, gmax), jnp.maximum(Ua, gmax))

        L, U = lax.fori_loop(
            0, NGROUPS, group_body,
            (jnp.full((BATCH,), jnp.inf, jnp.float32), neg_inf))

        def fill_body(s, _):
            cand[pl.ds(s * BATCH, BATCH)] = neg_inf
            return 0

        lax.fori_loop(0, CAP, fill_body, 0)

        def collect(col, ptr):
            v = gather_col(col)
            m = v >= L
            slot = jnp.minimum(ptr, CAP - 1)
            plsc.store_scatter(cand, [slot * BATCH + lanes], v, mask=m)
            return ptr + jnp.where(m, 1, 0).astype(jnp.int32)

        ptr = lax.fori_loop(0, latent_dim, collect,
                            jnp.zeros((BATCH,), jnp.int32))
        cmax = jnp.max(ptr)

        lo0 = _to_sortable(L)
        hi0 = _to_sortable(U) + jnp.uint32(1)

        def count_cand(mid_f):
            smax = jnp.minimum(cmax, CAP)

            def sbody(s, cnt):
                v = cand[pl.ds(s * BATCH, BATCH)]
                return cnt + jnp.where(v >= mid_f, 1, 0).astype(jnp.int32)

            return lax.fori_loop(0, smax, sbody,
                                 jnp.zeros((BATCH,), jnp.int32))

        def count_full(mid_f):
            def cbody(col, cnt):
                v = gather_col(col)
                return cnt + jnp.where(v >= mid_f, 1, 0).astype(jnp.int32)

            return lax.fori_loop(0, latent_dim, cbody,
                                 jnp.zeros((BATCH,), jnp.int32))

        t = lax.cond(
            cmax > CAP,
            lambda: bisect(lo0, hi0, count_full),
            lambda: bisect(lo0, hi0, count_cand),
        )

        tstage[pl.ds(b * BATCH, BATCH)] = t
        return 0

    lax.fori_loop(0, nbatch, process_batch, 0)
    pltpu.sync_copy(tstage, out_hbm.at[pl.ds(base, rows_per_worker)])


def kernel(x, b_pre, W_enc, b_enc, W_dec, b_dec):
    n, d = x.shape
    latent_dim = W_enc.shape[0]
    grid = (n // BLOCK_ROWS,)
    b_pre2 = b_pre.reshape(1, d)
    b_enc2 = b_enc.reshape(1, latent_dim)
    b_dec2 = b_dec.reshape(1, d)

    latents = pl.pallas_call(
        _enc_block,
        grid=grid,
        in_specs=[
            pl.BlockSpec((BLOCK_ROWS, d), lambda i: (i, 0)),
            pl.BlockSpec((1, d), lambda i: (0, 0)),
            pl.BlockSpec((latent_dim, d), lambda i: (0, 0)),
            pl.BlockSpec((1, latent_dim), lambda i: (0, 0)),
        ],
        out_specs=pl.BlockSpec((BLOCK_ROWS, latent_dim), lambda i: (i, 0)),
        out_shape=jax.ShapeDtypeStruct((n, latent_dim), jnp.float32),
    )(x, b_pre2, W_enc, b_enc2)

    mesh = plsc.VectorSubcoreMesh(core_axis_name="c", subcore_axis_name="s")
    rows_per_worker = n // NUM_WORKERS
    thresh = pl.kernel(
        _sc_thresh_body,
        mesh=mesh,
        out_type=jax.ShapeDtypeStruct((n,), jnp.float32),
        scratch_types=[
            pltpu.VMEM((BATCH * latent_dim,), jnp.float32),
            pltpu.VMEM((CAP * BATCH,), jnp.float32),
            pltpu.VMEM((rows_per_worker,), jnp.float32),
        ],
        compiler_params=pltpu.CompilerParams(needs_layout_passes=False),
    )(latents.reshape(-1))

    recon, sparse = pl.pallas_call(
        _dec_block,
        grid=grid,
        in_specs=[
            pl.BlockSpec((BLOCK_ROWS, latent_dim), lambda i: (i, 0)),
            pl.BlockSpec((BLOCK_ROWS, 1), lambda i: (i, 0)),
            pl.BlockSpec((1, d), lambda i: (0, 0)),
            pl.BlockSpec((d, latent_dim), lambda i: (0, 0)),
            pl.BlockSpec((1, d), lambda i: (0, 0)),
        ],
        out_specs=[
            pl.BlockSpec((BLOCK_ROWS, d), lambda i: (i, 0)),
            pl.BlockSpec((BLOCK_ROWS, latent_dim), lambda i: (i, 0)),
        ],
        out_shape=[
            jax.ShapeDtypeStruct((n, d), jnp.float32),
            jax.ShapeDtypeStruct((n, latent_dim), jnp.float32),
        ],
    )(latents, thresh.reshape(n, 1), b_pre2, W_dec, b_dec2)
    return (recon, sparse)
